# initial kernel scaffold (unmeasured)
import jax
import jax.numpy as jnp
from jax import lax
from jax.experimental import pallas as pl
from jax.experimental.pallas import tpu as pltpu


def kernel(x, W, labels):
    T, D = x.shape
    V_loc = W.shape[1]
    BV = 2048
    n_chunks = V_loc // BV

    def body(x_ref, W_ref, labels_ref, out_ref,
             m_ref, s_ref, ll_ref, comm_ref, recv_ref, send_sem, recv_sem):
        step = pl.program_id(0)

        @pl.when(step == 0)
        def _init():
            m_ref[:] = jnp.full((T,), -jnp.inf, dtype=jnp.float32)
            s_ref[:] = jnp.zeros((T,), dtype=jnp.float32)
            ll_ref[:] = jnp.zeros((T,), dtype=jnp.float32)

        logits = jnp.dot(x_ref[:], W_ref[:],
                         preferred_element_type=jnp.float32)

        m_old = m_ref[:]
        m_new = jnp.maximum(m_old, jnp.max(logits, axis=1))
        s_ref[:] = (s_ref[:] * jnp.exp(m_old - m_new)
                    + jnp.sum(jnp.exp(logits - m_new[:, None]), axis=1))
        m_ref[:] = m_new

        my_z = lax.axis_index("z")
        offset = my_z * V_loc + step * BV
        local_col = labels_ref[:] - offset
        cols = lax.broadcasted_iota(jnp.int32, (T, BV), 1)
        hit = cols == local_col[:, None]
        ll_ref[:] = ll_ref[:] + jnp.sum(jnp.where(hit, logits, 0.0), axis=1)

        @pl.when(step == n_chunks - 1)
        def _exchange():
            my_x = lax.axis_index("x")
            my_y = lax.axis_index("y")
            comm_ref[0, :] = m_ref[:]
            comm_ref[1, :] = s_ref[:]
            comm_ref[2, :] = ll_ref[:]
            rdma = pltpu.make_async_remote_copy(
                src_ref=comm_ref,
                dst_ref=recv_ref,
                send_sem=send_sem,
                recv_sem=recv_sem,
                device_id=(my_x, my_y, 1 - my_z),
                device_id_type=pl.DeviceIdType.MESH,
            )
            rdma.start()
            rdma.wait()

            m_me = m_ref[:]
            m_ot = recv_ref[0, :]
            m_g = jnp.maximum(m_me, m_ot)
            s_g = (s_ref[:] * jnp.exp(m_me - m_g)
                   + recv_ref[1, :] * jnp.exp(m_ot - m_g))
            lse = m_g + jnp.log(s_g)
            out_ref[:] = lse - (ll_ref[:] + recv_ref[2, :])

    return pl.pallas_call(
        body,
        grid=(n_chunks,),
        in_specs=[
            pl.BlockSpec(memory_space=pltpu.VMEM),
            pl.BlockSpec((D, BV), lambda i: (0, i)),
            pl.BlockSpec(memory_space=pltpu.VMEM),
        ],
        out_specs=pl.BlockSpec(memory_space=pltpu.VMEM),
        out_shape=jax.ShapeDtypeStruct((T,), jnp.float32),
        scratch_shapes=[
            pltpu.VMEM((T,), jnp.float32),
            pltpu.VMEM((T,), jnp.float32),
            pltpu.VMEM((T,), jnp.float32),
            pltpu.VMEM((3, T), jnp.float32),
            pltpu.VMEM((3, T), jnp.float32),
            pltpu.SemaphoreType.DMA,
            pltpu.SemaphoreType.DMA,
        ],
        compiler_params=pltpu.CompilerParams(
            dimension_semantics=("arbitrary",),
        ),
    )(x, W, labels)


# baseline (device time: 118823 ns/iter reference)
import jax
import jax.numpy as jnp
from jax import lax
from jax.experimental import pallas as pl
from jax.experimental.pallas import tpu as pltpu


def kernel(x, W, labels):
    T, D = x.shape
    V_loc = W.shape[1]
    BV = 2048
    n_chunks = V_loc // BV

    def body(x_ref, W_ref, labels_ref, out_ref,
             m_ref, s_ref, ll_ref, comm_ref, recv_ref, send_sem, recv_sem):
        step = pl.program_id(0)

        @pl.when(step == 0)
        def _init():
            m_ref[:] = jnp.full((T,), -jnp.inf, dtype=jnp.float32)
            s_ref[:] = jnp.zeros((T,), dtype=jnp.float32)
            ll_ref[:] = jnp.zeros((T,), dtype=jnp.float32)

        logits = jnp.dot(x_ref[:], W_ref[:],
                         preferred_element_type=jnp.float32)

        m_old = m_ref[:]
        m_new = jnp.maximum(m_old, jnp.max(logits, axis=1))
        s_ref[:] = (s_ref[:] * jnp.exp(m_old - m_new)
                    + jnp.sum(jnp.exp(logits - m_new[:, None]), axis=1))
        m_ref[:] = m_new

        my_z = lax.axis_index("z")
        offset = my_z * V_loc + step * BV
        local_col = labels_ref[:] - offset
        cols = lax.broadcasted_iota(jnp.int32, (T, BV), 1)
        hit = cols == local_col[:, None]
        ll_ref[:] = ll_ref[:] + jnp.sum(jnp.where(hit, logits, 0.0), axis=1)

        @pl.when(step == n_chunks - 1)
        def _exchange():
            my_x = lax.axis_index("x")
            my_y = lax.axis_index("y")
            comm_ref[0, :] = m_ref[:]
            comm_ref[1, :] = s_ref[:]
            comm_ref[2, :] = ll_ref[:]
            rdma = pltpu.make_async_remote_copy(
                src_ref=comm_ref,
                dst_ref=recv_ref,
                send_sem=send_sem,
                recv_sem=recv_sem,
                device_id=(my_x, my_y, 1 - my_z),
                device_id_type=pl.DeviceIdType.MESH,
            )
            rdma.start()
            rdma.wait()

            m_me = m_ref[:]
            m_ot = recv_ref[0, :]
            m_g = jnp.maximum(m_me, m_ot)
            s_g = (s_ref[:] * jnp.exp(m_me - m_g)
                   + recv_ref[1, :] * jnp.exp(m_ot - m_g))
            lse = m_g + jnp.log(s_g)
            out_ref[:] = lse - (ll_ref[:] + recv_ref[2, :])

    return pl.pallas_call(
        body,
        grid=(n_chunks,),
        in_specs=[
            pl.BlockSpec(memory_space=pltpu.VMEM),
            pl.BlockSpec((D, BV), lambda i: (0, i)),
            pl.BlockSpec(memory_space=pltpu.VMEM),
        ],
        out_specs=pl.BlockSpec(memory_space=pltpu.VMEM),
        out_shape=jax.ShapeDtypeStruct((T,), jnp.float32),
        scratch_shapes=[
            pltpu.VMEM((T,), jnp.float32),
            pltpu.VMEM((T,), jnp.float32),
            pltpu.VMEM((T,), jnp.float32),
            pltpu.VMEM((3, T), jnp.float32),
            pltpu.VMEM((3, T), jnp.float32),
            pltpu.SemaphoreType.DMA,
            pltpu.SemaphoreType.DMA,
        ],
        compiler_params=pltpu.CompilerParams(
            dimension_semantics=("arbitrary",),
            vmem_limit_bytes=100 * 1024 * 1024,
        ),
    )(x, W, labels)
